# baseline (speedup 1.0000x reference)
import functools
import math

import jax
import jax.numpy as jnp
from jax import lax
from jax.experimental import pallas as pl
from jax.experimental.pallas import tpu as pltpu
from jax.experimental.pallas import tpu_sc as plsc

D_MODEL = 64
SCALE = math.sqrt(D_MODEL)
NC, NS = 2, 16
NW = NC * NS
GRP = 128
CHUNK = 1024
G = CHUNK // GRP
LANES = 16


def _emb_body(idx_hbm, table_hbm, out_hbm, idx_v, rows_v, sem):
    wid = lax.axis_index("s") * NC + lax.axis_index("c")
    n_rows = out_hbm.shape[0]
    per_w = n_rows // NW
    n_chunks = per_w // CHUNK
    base = wid * per_w

    def chunk_body(ci, carry):
        rb = base + ci * CHUNK
        irb = pl.multiple_of(rb // GRP, 8)
        pltpu.sync_copy(idx_hbm.at[pl.ds(irb, G), :], idx_v)
        copies = [
            pltpu.async_copy(
                table_hbm.at[idx_v.at[j]],
                rows_v.at[pl.ds(j * GRP, GRP), :],
                sem,
            )
            for j in range(G)
        ]
        for c in copies:
            c.wait()

        def scale_row(r, c2):
            for col in range(D_MODEL // LANES):
                sl = pl.ds(col * LANES, LANES)
                rows_v[r, sl] = rows_v[r, sl] * SCALE
            return c2

        lax.fori_loop(0, CHUNK, scale_row, 0)
        pltpu.sync_copy(rows_v, out_hbm.at[pl.ds(rb, CHUNK), :])
        return carry

    lax.fori_loop(0, n_chunks, chunk_body, 0)


def kernel(X, table):
    n = X.shape[0] * X.shape[1]
    idx2d = X.reshape(n // GRP, GRP).astype(jnp.int32)
    mesh = plsc.VectorSubcoreMesh(core_axis_name="c", subcore_axis_name="s")
    run = pl.kernel(
        _emb_body,
        out_type=jax.ShapeDtypeStruct((n, D_MODEL), jnp.float32),
        mesh=mesh,
        compiler_params=pltpu.CompilerParams(use_tc_tiling_on_sc=False),
        scratch_types=[
            pltpu.VMEM((G, GRP), jnp.int32),
            pltpu.VMEM((CHUNK, D_MODEL), jnp.float32),
            pltpu.SemaphoreType.DMA,
        ],
    )
    out = run(idx2d, table)
    return out.reshape(X.shape[0], X.shape[1], D_MODEL)


# speedup vs baseline: 1.0792x; 1.0792x over previous
import math

import jax
import jax.numpy as jnp
from jax import lax
from jax.experimental import pallas as pl
from jax.experimental.pallas import tpu as pltpu
from jax.experimental.pallas import tpu_sc as plsc

D_MODEL = 64
SCALE = math.sqrt(D_MODEL)
NC, NS = 2, 16
NW = NC * NS
GRP = 128
HALF = 512
HG = HALF // GRP
LANES = 16


def _fire_gather(table_hbm, idx_all, rows_b, gsem, h):
    jbase = h * HG
    for k in range(HG):
        pltpu.async_copy(
            table_hbm.at[idx_all.at[jbase + k]],
            rows_b.at[pl.ds(k * GRP, GRP), :],
            gsem,
        )


def _wait_gather(table_hbm, rows_b, gsem):
    for k in range(HG):
        pltpu.make_async_copy(
            table_hbm.at[pl.ds(0, GRP), :],
            rows_b.at[pl.ds(k * GRP, GRP), :],
            gsem,
        ).wait()


def _scale_half(rows_b):
    def scale_row(r, c2):
        for col in range(D_MODEL // LANES):
            sl = pl.ds(col * LANES, LANES)
            rows_b[r, sl] = rows_b[r, sl] * SCALE
        return c2

    lax.fori_loop(0, HALF, scale_row, 0)


def _emb_body(idx_hbm, table_hbm, out_hbm, idx_all, rows0, rows1, g0, g1,
              o0, o1):
    wid = lax.axis_index("s") * NC + lax.axis_index("c")
    n_rows = out_hbm.shape[0]
    per_w = n_rows // NW
    n_steps = per_w // (2 * HALF)
    base = wid * per_w
    ibase = pl.multiple_of(wid * (per_w // GRP), 8)

    pltpu.sync_copy(idx_hbm.at[pl.ds(ibase, per_w // GRP), :], idx_all)
    _fire_gather(table_hbm, idx_all, rows0, g0, 0)

    def _wait_out(rows_b, osem):
        pltpu.make_async_copy(
            rows_b, out_hbm.at[pl.ds(0, HALF), :], osem
        ).wait()

    def step(s, carry):
        h0 = 2 * s

        @pl.when(s > 0)
        def _():
            _wait_out(rows1, o1)

        _fire_gather(table_hbm, idx_all, rows1, g1, h0 + 1)
        _wait_gather(table_hbm, rows0, g0)
        _scale_half(rows0)
        pltpu.async_copy(rows0, out_hbm.at[pl.ds(base + h0 * HALF, HALF), :],
                         o0)

        @pl.when(s < n_steps - 1)
        def _():
            _wait_out(rows0, o0)
            _fire_gather(table_hbm, idx_all, rows0, g0, h0 + 2)

        _wait_gather(table_hbm, rows1, g1)
        _scale_half(rows1)
        pltpu.async_copy(
            rows1, out_hbm.at[pl.ds(base + (h0 + 1) * HALF, HALF), :], o1)
        return carry

    lax.fori_loop(0, n_steps, step, 0)
    _wait_out(rows0, o0)
    _wait_out(rows1, o1)


def kernel(X, table):
    n = X.shape[0] * X.shape[1]
    idx2d = X.reshape(n // GRP, GRP).astype(jnp.int32)
    per_w = n // NW
    mesh = plsc.VectorSubcoreMesh(core_axis_name="c", subcore_axis_name="s")
    run = pl.kernel(
        _emb_body,
        out_type=jax.ShapeDtypeStruct((n, D_MODEL), jnp.float32),
        mesh=mesh,
        compiler_params=pltpu.CompilerParams(use_tc_tiling_on_sc=False),
        scratch_types=[
            pltpu.VMEM((per_w // GRP, GRP), jnp.int32),
            pltpu.VMEM((HALF, D_MODEL), jnp.float32),
            pltpu.VMEM((HALF, D_MODEL), jnp.float32),
            pltpu.SemaphoreType.DMA,
            pltpu.SemaphoreType.DMA,
            pltpu.SemaphoreType.DMA,
            pltpu.SemaphoreType.DMA,
        ],
    )
    out = run(idx2d, table)
    return out.reshape(X.shape[0], X.shape[1], D_MODEL)
